# baseline (speedup 1.0000x reference)
import functools

import jax
import jax.numpy as jnp
from jax import lax
from jax.experimental import pallas as pl
from jax.experimental.pallas import tpu as pltpu
from jax.experimental.pallas import tpu_sc as plsc

VOCAB = 1000000
EMBED = 32
BATCH = 4096
SEQ = 200

_info = plsc.get_sparse_core_info()
NC = _info.num_cores
NS = _info.num_subcores
NW = NC * NS

SB = SEQ // 8
CB = BATCH // 128
FB = EMBED // 8
TI = 128
L = 16


def _sc_gather(idxp, table):
    mesh = plsc.VectorSubcoreMesh(core_axis_name="c", subcore_axis_name="s")

    @functools.partial(
        pl.kernel,
        mesh=mesh,
        out_type=jax.ShapeDtypeStruct((SEQ, FB, CB, 8, TI), jnp.float32),
        scratch_types=[
            pltpu.VMEM((2, TI), jnp.int32),
            pltpu.VMEM((2, TI, EMBED), jnp.float32),
            pltpu.VMEM((2, FB, 8, TI), jnp.float32),
            pltpu.SemaphoreType.DMA((2,)),
            pltpu.SemaphoreType.DMA((2,)),
            pltpu.SemaphoreType.DMA((2,)),
        ],
        compiler_params=pltpu.CompilerParams(
            use_tc_tiling_on_sc=False, needs_layout_passes=False),
    )
    def k(idx_hbm, table_hbm, out_hbm, idx_v, rows_v, obuf, isem, gsem, wsem):
        w = lax.axis_index("s") * NC + lax.axis_index("c")

        def idx_load(s, b):
            pltpu.async_copy(
                idx_hbm.at[s // 8, w, s % 8], idx_v.at[b], isem.at[b])

        def transpose_round(b):
            rows = rows_v.at[b]
            for f in range(EMBED):
                col = jnp.full((L,), f, jnp.int32)
                for t0 in range(TI // L):
                    row = lax.iota(jnp.int32, L) + (t0 * L)
                    vals = plsc.load_gather(rows, [row, col])
                    obuf[b, f // 8, f % 8, pl.ds(t0 * L, L)] = vals

        idx_load(0, 0)
        idx_load(1, 1)

        def body(g, carry):
            for b in range(2):
                s = g * 2 + b
                bp = 1 - b
                pltpu.make_async_copy(
                    idx_hbm.at[s // 8, w, s % 8], idx_v.at[b], isem.at[b]).wait()
                pltpu.async_copy(
                    table_hbm.at[idx_v.at[b]], rows_v.at[b], gsem.at[b])

                @pl.when(s >= 1)
                def _():
                    pltpu.make_async_copy(
                        table_hbm.at[idx_v.at[bp]], rows_v.at[bp],
                        gsem.at[bp]).wait()

                    @pl.when(s + 1 < SEQ)
                    def _():
                        idx_load(s + 1, bp)

                    @pl.when(s >= 3)
                    def _():
                        pltpu.make_async_copy(
                            obuf.at[bp], out_hbm.at[s - 3, :, w],
                            wsem.at[bp]).wait()

                    transpose_round(bp)
                    pltpu.async_copy(
                        obuf.at[bp], out_hbm.at[s - 1, :, w], wsem.at[bp])
            return carry

        lax.fori_loop(0, SEQ // 2, body, 0)

        bl = 1
        pltpu.make_async_copy(
            table_hbm.at[idx_v.at[bl]], rows_v.at[bl], gsem.at[bl]).wait()
        pltpu.make_async_copy(
            obuf.at[bl], out_hbm.at[SEQ - 3, :, w], wsem.at[bl]).wait()
        transpose_round(bl)
        pltpu.async_copy(obuf.at[bl], out_hbm.at[SEQ - 1, :, w], wsem.at[bl])
        pltpu.make_async_copy(
            obuf.at[0], out_hbm.at[SEQ - 2, :, w], wsem.at[0]).wait()
        pltpu.make_async_copy(
            obuf.at[bl], out_hbm.at[SEQ - 1, :, w], wsem.at[bl]).wait()

    return k(idxp, table)


def kernel(token_ids, table):
    idxp = token_ids.T.reshape(SB, 8, CB, TI).transpose(0, 2, 1, 3)
    o = _sc_gather(idxp, table)
    return o.transpose(2, 4, 0, 1, 3).reshape(BATCH, SEQ, EMBED)


# speedup vs baseline: 1.3241x; 1.3241x over previous
import functools

import jax
import jax.numpy as jnp
from jax import lax
from jax.experimental import pallas as pl
from jax.experimental.pallas import tpu as pltpu
from jax.experimental.pallas import tpu_sc as plsc

VOCAB = 1000000
EMBED = 32
BATCH = 4096
SEQ = 200

_info = plsc.get_sparse_core_info()
NC = _info.num_cores
NS = _info.num_subcores
NW = NC * NS

SB = SEQ // 8
CB = BATCH // 128
FB = EMBED // 8
TI = 128
L = 16
SG = 4
NR = SEQ // SG


def _sc_gather(idxp, table):
    mesh = plsc.VectorSubcoreMesh(core_axis_name="c", subcore_axis_name="s")

    @functools.partial(
        pl.kernel,
        mesh=mesh,
        out_type=jax.ShapeDtypeStruct((SEQ, FB, CB, 8, TI), jnp.float32),
        scratch_types=[
            pltpu.VMEM((2, SG, TI), jnp.int32),
            pltpu.VMEM((2, SG, TI, EMBED), jnp.float32),
            pltpu.VMEM((2, SG, FB, 8, TI), jnp.float32),
            pltpu.SemaphoreType.DMA((2,)),
            pltpu.SemaphoreType.DMA((2,)),
            pltpu.SemaphoreType.DMA((2,)),
        ],
        compiler_params=pltpu.CompilerParams(
            use_tc_tiling_on_sc=False, needs_layout_passes=False),
    )
    def k(idx_hbm, table_hbm, out_hbm, idx_v, rows_v, obuf, isem, gsem, wsem):
        w = lax.axis_index("s") * NC + lax.axis_index("c")

        def idx_load(r, b):
            pltpu.async_copy(
                idx_hbm.at[r // 2, w, pl.ds((r % 2) * SG, SG)],
                idx_v.at[b], isem.at[b])

        def idx_wait(r, b):
            pltpu.make_async_copy(
                idx_hbm.at[r // 2, w, pl.ds((r % 2) * SG, SG)],
                idx_v.at[b], isem.at[b]).wait()

        def gathers_start(b):
            for si in range(SG):
                pltpu.async_copy(
                    table_hbm.at[idx_v.at[b, si]], rows_v.at[b, si],
                    gsem.at[b])

        def gathers_wait(b):
            for si in range(SG):
                pltpu.make_async_copy(
                    table_hbm.at[idx_v.at[b, si]], rows_v.at[b, si],
                    gsem.at[b]).wait()

        def wb_start(r, b):
            pltpu.async_copy(
                obuf.at[b], out_hbm.at[pl.ds(r * SG, SG), :, w], wsem.at[b])

        def wb_wait(r, b):
            pltpu.make_async_copy(
                obuf.at[b], out_hbm.at[pl.ds(r * SG, SG), :, w],
                wsem.at[b]).wait()

        def transpose_round(b):
            for si in range(SG):
                rows = rows_v.at[b, si]
                for t0 in range(TI // L):
                    row = lax.iota(jnp.int32, L) + (t0 * L)
                    vals = [
                        plsc.load_gather(rows, [row, jnp.full((L,), f, jnp.int32)])
                        for f in range(EMBED)
                    ]
                    for f in range(EMBED):
                        obuf[b, si, f // 8, f % 8, pl.ds(t0 * L, L)] = vals[f]

        idx_load(0, 0)
        idx_load(1, 1)

        def body(g, carry):
            for b in range(2):
                r = g * 2 + b
                bp = 1 - b
                idx_wait(r, b)
                gathers_start(b)

                @pl.when(r >= 1)
                def _():
                    gathers_wait(bp)

                    @pl.when(r + 1 < NR)
                    def _():
                        idx_load(r + 1, bp)

                    @pl.when(r >= 3)
                    def _():
                        wb_wait(r - 3, bp)

                    transpose_round(bp)
                    wb_start(r - 1, bp)
            return carry

        lax.fori_loop(0, NR // 2, body, 0)

        gathers_wait(1)
        wb_wait(NR - 3, 1)
        transpose_round(1)
        wb_start(NR - 1, 1)
        wb_wait(NR - 2, 0)
        wb_wait(NR - 1, 1)

    return k(idxp, table)


def kernel(token_ids, table):
    idxp = token_ids.T.reshape(SB, 8, CB, TI).transpose(0, 2, 1, 3)
    o = _sc_gather(idxp, table)
    return o.transpose(2, 4, 0, 1, 3).reshape(BATCH, SEQ, EMBED)
